# baseline (speedup 1.0000x reference)
import functools

import jax
import jax.numpy as jnp
from jax.experimental import pallas as pl
from jax.experimental.pallas import tpu as pltpu
from jax.experimental.pallas import tpu_sc as plsc

E = 8
TOPK = 2
D = 1024
DFF = 704
NSH = 2
T = 2048
RSF = 2.5
EPAD = 128

TM_C = 256
P = T * TOPK + E * TM_C
NT = P // TM_C
TME = 512

_SC_NC = 2
_SC_NS = 16
NW = _SC_NC * _SC_NS
TPW = T // NW


def _router_body(x_ref, gwp_ref, biasp_ref,
                 pos0_ref, pos1_ref, w0_ref, w1_ref, gid_ref):
    x = x_ref[...]
    logits = jax.lax.dot_general(
        x, gwp_ref[...], (((1,), (1,)), ((), ())),
        preferred_element_type=jnp.float32)
    s = jax.nn.sigmoid(logits)
    sel = s + biasp_ref[...]
    m1 = jnp.max(sel, axis=1, keepdims=True)
    t1 = sel >= m1
    sel2 = jnp.where(t1, -jnp.inf, sel)
    m2 = jnp.max(sel2, axis=1, keepdims=True)
    t2 = sel2 >= m2
    t1f = t1.astype(jnp.float32)
    t2f = t2.astype(jnp.float32)

    denom = jnp.sum(s * (t1f + t2f), axis=1, keepdims=True) + 1e-20
    w0_ref[...] = jnp.sum(s * t1f, axis=1, keepdims=True) * RSF / denom
    w1_ref[...] = jnp.sum(s * t2f, axis=1, keepdims=True) * RSF / denom

    cnt = t1f + t2f
    inc = cnt
    sft = 1
    while sft < T:
        inc = inc + jnp.concatenate(
            [jnp.zeros((sft, EPAD), jnp.float32), inc[:T - sft, :]], axis=0)
        sft *= 2
    exc = inc - cnt
    totals = inc[T - 1:T, :]

    tot_i = totals.astype(jnp.int32)
    lane = jax.lax.broadcasted_iota(jnp.int32, (1, EPAD), 1)
    jv = lane * TM_C
    offs = jnp.zeros((1, EPAD), jnp.float32)
    gid = jnp.zeros((1, EPAD), jnp.int32)
    run = jnp.zeros((), jnp.int32)
    for e in range(E):
        oh = lane == e
        te = jnp.sum(jnp.where(oh, tot_i, 0))
        pe = ((te + TM_C - 1) // TM_C) * TM_C
        offs = offs + jnp.where(oh, run, 0).astype(jnp.float32)
        run = run + pe
        gid = gid + (jv >= run).astype(jnp.int32)
    gid_ref[...] = jnp.minimum(gid, E - 1)

    dest = exc + offs
    pos0_ref[...] = jnp.sum(dest * t1f, axis=1, keepdims=True).astype(jnp.int32)
    pos1_ref[...] = jnp.sum(dest * t2f, axis=1, keepdims=True).astype(jnp.int32)


def _router(x, gwp, biasp):
    return pl.pallas_call(
        _router_body,
        out_shape=[
            jax.ShapeDtypeStruct((T, 1), jnp.int32),
            jax.ShapeDtypeStruct((T, 1), jnp.int32),
            jax.ShapeDtypeStruct((T, 1), jnp.float32),
            jax.ShapeDtypeStruct((T, 1), jnp.float32),
            jax.ShapeDtypeStruct((1, EPAD), jnp.int32),
        ],
    )(x, gwp, biasp)


def _dispatch_body(x_hbm, p0_hbm, p1_hbm, xs_hbm, idx_v, xv, sem):
    w = jax.lax.axis_index("s") * _SC_NC + jax.lax.axis_index("c")
    base = w * TPW
    pltpu.sync_copy(p0_hbm.at[pl.ds(base, TPW)], idx_v.at[0])
    pltpu.sync_copy(p1_hbm.at[pl.ds(base, TPW)], idx_v.at[1])
    pltpu.sync_copy(x_hbm.at[pl.ds(base, TPW)], xv)
    pltpu.async_copy(xv, xs_hbm.at[idx_v.at[0]], sem).wait()
    pltpu.async_copy(xv, xs_hbm.at[idx_v.at[1]], sem).wait()


@functools.lru_cache(maxsize=None)
def _make_dispatch():
    return pl.kernel(
        _dispatch_body,
        out_type=jax.ShapeDtypeStruct((P, D), jnp.float32),
        mesh=plsc.VectorSubcoreMesh(core_axis_name="c", subcore_axis_name="s"),
        scratch_types=[
            pltpu.VMEM((2, TPW), jnp.int32),
            pltpu.VMEM((TPW, D), jnp.float32),
            pltpu.SemaphoreType.DMA,
        ],
    )


def _dispatch(x, p0, p1):
    return _make_dispatch()(x, p0, p1)


def _gmm_body(gid_ref, xs_ref, w13_ref, w2_ref, ys_ref):
    xb = xs_ref[...].astype(jnp.bfloat16)
    gu = jax.lax.dot_general(
        xb, w13_ref[0], (((1,), (1,)), ((), ())),
        preferred_element_type=jnp.float32)
    g = gu[:, :DFF]
    u = gu[:, DFF:]
    h = (g * jax.nn.sigmoid(g) * u).astype(jnp.bfloat16)
    ys_ref[...] = jax.lax.dot_general(
        h, w2_ref[0], (((1,), (1,)), ((), ())),
        preferred_element_type=jnp.float32)


def _gmm(gid, xs, w13b, w2b):
    grid_spec = pltpu.PrefetchScalarGridSpec(
        num_scalar_prefetch=1,
        grid=(NT,),
        in_specs=[
            pl.BlockSpec((TM_C, D), lambda j, gid: (j, 0)),
            pl.BlockSpec((1, 2 * DFF, D), lambda j, gid: (gid[j], 0, 0)),
            pl.BlockSpec((1, D, DFF), lambda j, gid: (gid[j], 0, 0)),
        ],
        out_specs=pl.BlockSpec((TM_C, D), lambda j, gid: (j, 0)),
    )
    return pl.pallas_call(
        _gmm_body,
        grid_spec=grid_spec,
        out_shape=jax.ShapeDtypeStruct((P, D), jnp.float32),
        compiler_params=pltpu.CompilerParams(
            dimension_semantics=("arbitrary",),
        ),
    )(gid, xs, w13b, w2b)


def _gatherz_body(ys_hbm, p0_hbm, p1_hbm, z0_hbm, z1_hbm, idx_v, rv, sem):
    w = jax.lax.axis_index("s") * _SC_NC + jax.lax.axis_index("c")
    base = w * TPW
    pltpu.sync_copy(p0_hbm.at[pl.ds(base, TPW)], idx_v.at[0])
    pltpu.sync_copy(p1_hbm.at[pl.ds(base, TPW)], idx_v.at[1])
    pltpu.async_copy(ys_hbm.at[idx_v.at[0]], rv, sem).wait()
    pltpu.sync_copy(rv, z0_hbm.at[pl.ds(base, TPW)])
    pltpu.async_copy(ys_hbm.at[idx_v.at[1]], rv, sem).wait()
    pltpu.sync_copy(rv, z1_hbm.at[pl.ds(base, TPW)])


@functools.lru_cache(maxsize=None)
def _make_gatherz():
    return pl.kernel(
        _gatherz_body,
        out_type=(
            jax.ShapeDtypeStruct((T, D), jnp.float32),
            jax.ShapeDtypeStruct((T, D), jnp.float32),
        ),
        mesh=plsc.VectorSubcoreMesh(core_axis_name="c", subcore_axis_name="s"),
        scratch_types=[
            pltpu.VMEM((2, TPW), jnp.int32),
            pltpu.VMEM((TPW, D), jnp.float32),
            pltpu.SemaphoreType.DMA,
        ],
    )


def _gatherz(ys, p0, p1):
    return _make_gatherz()(ys, p0, p1)


def _shared_body(x_ref, sgu_ref, sdn_ref, out_ref):
    xb = x_ref[...].astype(jnp.bfloat16)
    sgu = jax.lax.dot_general(
        xb, sgu_ref[...], (((1,), (1,)), ((), ())),
        preferred_element_type=jnp.float32)
    sg = sgu[:, :DFF * NSH]
    su = sgu[:, DFF * NSH:]
    sh = (sg * jax.nn.sigmoid(sg) * su).astype(jnp.bfloat16)
    out_ref[...] = jax.lax.dot_general(
        sh, sdn_ref[...], (((1,), (1,)), ((), ())),
        preferred_element_type=jnp.float32)


def _shared_half(xh, sgub, sdnb):
    return pl.pallas_call(
        _shared_body,
        grid=(T // 2 // TME,),
        in_specs=[
            pl.BlockSpec((TME, D), lambda i: (i, 0)),
            pl.BlockSpec((2 * DFF * NSH, D), lambda i: (0, 0)),
            pl.BlockSpec((D, DFF * NSH), lambda i: (0, 0)),
        ],
        out_specs=pl.BlockSpec((TME, D), lambda i: (i, 0)),
        out_shape=jax.ShapeDtypeStruct((T // 2, D), jnp.float32),
        compiler_params=pltpu.CompilerParams(
            dimension_semantics=("arbitrary",),
        ),
    )(xh, sgub, sdnb)


def _combine_body(z0_ref, z1_ref, w0_ref, w1_ref, sh_ref, out_ref):
    out_ref[...] = (w0_ref[...] * z0_ref[...] + w1_ref[...] * z1_ref[...]
                    + sh_ref[...])


def _combine(z0, z1, w0, w1, shared_out):
    return pl.pallas_call(
        _combine_body,
        grid=(T // TME,),
        in_specs=[
            pl.BlockSpec((TME, D), lambda i: (i, 0)),
            pl.BlockSpec((TME, D), lambda i: (i, 0)),
            pl.BlockSpec((TME, 1), lambda i: (i, 0)),
            pl.BlockSpec((TME, 1), lambda i: (i, 0)),
            pl.BlockSpec((TME, D), lambda i: (i, 0)),
        ],
        out_specs=pl.BlockSpec((TME, D), lambda i: (i, 0)),
        out_shape=jax.ShapeDtypeStruct((T, D), jnp.float32),
        compiler_params=pltpu.CompilerParams(
            dimension_semantics=("arbitrary",),
        ),
    )(z0, z1, w0, w1, shared_out)


def kernel(hidden_states, residual, gate_weight, e_score_correction_bias,
           w13, w2, shared_gate_up, shared_down):
    del residual
    x = hidden_states
    gwp = jnp.zeros((EPAD, D), jnp.float32).at[:E].set(gate_weight)
    biasp = jnp.full((1, EPAD), -1e30, jnp.float32
                     ).at[0, :E].set(e_score_correction_bias)

    pos0, pos1, w0, w1, gid = _router(x, gwp, biasp)
    p0 = pos0.reshape(T)
    p1 = pos1.reshape(T)

    sgub = shared_gate_up.astype(jnp.bfloat16)
    sdnb = shared_down.astype(jnp.bfloat16)

    xs = _dispatch(x, p0, p1)
    sh_a = _shared_half(x[:T // 2], sgub, sdnb)
    ys = _gmm(gid.reshape(EPAD), xs,
              w13.astype(jnp.bfloat16), w2.astype(jnp.bfloat16))
    z0, z1 = _gatherz(ys, p0, p1)
    sh_b = _shared_half(x[T // 2:], sgub, sdnb)
    shared_out = jnp.concatenate([sh_a, sh_b], axis=0)

    return _combine(z0, z1, w0, w1, shared_out)


# speedup vs baseline: 1.1228x; 1.1228x over previous
import functools

import jax
import jax.numpy as jnp
from jax.experimental import pallas as pl
from jax.experimental.pallas import tpu as pltpu
from jax.experimental.pallas import tpu_sc as plsc

E = 8
TOPK = 2
D = 1024
DFF = 704
NSH = 2
T = 2048
RSF = 2.5
EPAD = 128

TM_C = 256
P = T * TOPK + E * TM_C
NT = P // TM_C
TME = 512

_SC_NC = 2
_SC_NS = 16
NW = _SC_NC * _SC_NS
TPW = T // NW


def _router_body(x_ref, gwp_ref, biasp_ref,
                 pos0_ref, pos1_ref, w0_ref, w1_ref, gid_ref):
    x = x_ref[...]
    logits = jax.lax.dot_general(
        x, gwp_ref[...], (((1,), (1,)), ((), ())),
        preferred_element_type=jnp.float32)
    s = jax.nn.sigmoid(logits)
    sel = s + biasp_ref[...]
    m1 = jnp.max(sel, axis=1, keepdims=True)
    t1 = sel >= m1
    sel2 = jnp.where(t1, -jnp.inf, sel)
    m2 = jnp.max(sel2, axis=1, keepdims=True)
    t2 = sel2 >= m2
    t1f = t1.astype(jnp.float32)
    t2f = t2.astype(jnp.float32)

    denom = jnp.sum(s * (t1f + t2f), axis=1, keepdims=True) + 1e-20
    w0_ref[...] = jnp.sum(s * t1f, axis=1, keepdims=True) * RSF / denom
    w1_ref[...] = jnp.sum(s * t2f, axis=1, keepdims=True) * RSF / denom

    cnt = t1f + t2f
    inc = cnt
    sft = 1
    while sft < T:
        inc = inc + jnp.concatenate(
            [jnp.zeros((sft, EPAD), jnp.float32), inc[:T - sft, :]], axis=0)
        sft *= 2
    exc = inc - cnt
    totals = inc[T - 1:T, :]

    tot_i = totals.astype(jnp.int32)
    lane = jax.lax.broadcasted_iota(jnp.int32, (1, EPAD), 1)
    jv = lane * TM_C
    offs = jnp.zeros((1, EPAD), jnp.float32)
    gid = jnp.zeros((1, EPAD), jnp.int32)
    run = jnp.zeros((), jnp.int32)
    for e in range(E):
        oh = lane == e
        te = jnp.sum(jnp.where(oh, tot_i, 0))
        pe = ((te + TM_C - 1) // TM_C) * TM_C
        offs = offs + jnp.where(oh, run, 0).astype(jnp.float32)
        run = run + pe
        gid = gid + (jv >= run).astype(jnp.int32)
    gid_ref[...] = jnp.minimum(gid, E - 1)

    dest = exc + offs
    pos0_ref[...] = jnp.sum(dest * t1f, axis=1, keepdims=True).astype(jnp.int32)
    pos1_ref[...] = jnp.sum(dest * t2f, axis=1, keepdims=True).astype(jnp.int32)


def _router(x, gwp, biasp):
    return pl.pallas_call(
        _router_body,
        out_shape=[
            jax.ShapeDtypeStruct((T, 1), jnp.int32),
            jax.ShapeDtypeStruct((T, 1), jnp.int32),
            jax.ShapeDtypeStruct((T, 1), jnp.float32),
            jax.ShapeDtypeStruct((T, 1), jnp.float32),
            jax.ShapeDtypeStruct((1, EPAD), jnp.int32),
        ],
    )(x, gwp, biasp)


def _dispatch_body(x_hbm, p0_hbm, p1_hbm, xs_hbm, idx_v, xv, sem):
    w = jax.lax.axis_index("s") * _SC_NC + jax.lax.axis_index("c")
    base = w * TPW
    pltpu.sync_copy(p0_hbm.at[pl.ds(base, TPW)], idx_v.at[0])
    pltpu.sync_copy(p1_hbm.at[pl.ds(base, TPW)], idx_v.at[1])
    pltpu.sync_copy(x_hbm.at[pl.ds(base, TPW)], xv)
    pltpu.async_copy(xv, xs_hbm.at[idx_v.at[0]], sem).wait()
    pltpu.async_copy(xv, xs_hbm.at[idx_v.at[1]], sem).wait()


@functools.lru_cache(maxsize=None)
def _make_dispatch():
    return pl.kernel(
        _dispatch_body,
        out_type=jax.ShapeDtypeStruct((P, D), jnp.float32),
        mesh=plsc.VectorSubcoreMesh(core_axis_name="c", subcore_axis_name="s"),
        scratch_types=[
            pltpu.VMEM((2, TPW), jnp.int32),
            pltpu.VMEM((TPW, D), jnp.float32),
            pltpu.SemaphoreType.DMA,
        ],
    )


def _dispatch(x, p0, p1):
    return _make_dispatch()(x, p0, p1)


def _gmm_body(gid_ref, xs_ref, w13_ref, w2_ref, ys_ref):
    xb = xs_ref[...].astype(jnp.bfloat16)
    gu = jax.lax.dot_general(
        xb, w13_ref[0].astype(jnp.bfloat16), (((1,), (1,)), ((), ())),
        preferred_element_type=jnp.float32)
    g = gu[:, :DFF]
    u = gu[:, DFF:]
    h = (g * jax.nn.sigmoid(g) * u).astype(jnp.bfloat16)
    ys_ref[...] = jax.lax.dot_general(
        h, w2_ref[0].astype(jnp.bfloat16), (((1,), (1,)), ((), ())),
        preferred_element_type=jnp.float32)


def _gmm(gid, xs, w13b, w2b):
    grid_spec = pltpu.PrefetchScalarGridSpec(
        num_scalar_prefetch=1,
        grid=(NT,),
        in_specs=[
            pl.BlockSpec((TM_C, D), lambda j, gid: (j, 0)),
            pl.BlockSpec((1, 2 * DFF, D), lambda j, gid: (gid[j], 0, 0)),
            pl.BlockSpec((1, D, DFF), lambda j, gid: (gid[j], 0, 0)),
        ],
        out_specs=pl.BlockSpec((TM_C, D), lambda j, gid: (j, 0)),
    )
    return pl.pallas_call(
        _gmm_body,
        grid_spec=grid_spec,
        out_shape=jax.ShapeDtypeStruct((P, D), jnp.float32),
        compiler_params=pltpu.CompilerParams(
            dimension_semantics=("arbitrary",),
        ),
    )(gid, xs, w13b, w2b)


def _gatherz_body(ys_hbm, p0_hbm, p1_hbm, z0_hbm, z1_hbm, idx_v, rv, sem):
    w = jax.lax.axis_index("s") * _SC_NC + jax.lax.axis_index("c")
    base = w * TPW
    pltpu.sync_copy(p0_hbm.at[pl.ds(base, TPW)], idx_v.at[0])
    pltpu.sync_copy(p1_hbm.at[pl.ds(base, TPW)], idx_v.at[1])
    pltpu.async_copy(ys_hbm.at[idx_v.at[0]], rv, sem).wait()
    pltpu.sync_copy(rv, z0_hbm.at[pl.ds(base, TPW)])
    pltpu.async_copy(ys_hbm.at[idx_v.at[1]], rv, sem).wait()
    pltpu.sync_copy(rv, z1_hbm.at[pl.ds(base, TPW)])


@functools.lru_cache(maxsize=None)
def _make_gatherz():
    return pl.kernel(
        _gatherz_body,
        out_type=(
            jax.ShapeDtypeStruct((T, D), jnp.float32),
            jax.ShapeDtypeStruct((T, D), jnp.float32),
        ),
        mesh=plsc.VectorSubcoreMesh(core_axis_name="c", subcore_axis_name="s"),
        scratch_types=[
            pltpu.VMEM((2, TPW), jnp.int32),
            pltpu.VMEM((TPW, D), jnp.float32),
            pltpu.SemaphoreType.DMA,
        ],
    )


def _gatherz(ys, p0, p1):
    return _make_gatherz()(ys, p0, p1)


def _shared_body(x_ref, sgu_ref, sdn_ref, out_ref):
    xb = x_ref[...].astype(jnp.bfloat16)
    sgu = jax.lax.dot_general(
        xb, sgu_ref[...].astype(jnp.bfloat16), (((1,), (1,)), ((), ())),
        preferred_element_type=jnp.float32)
    sg = sgu[:, :DFF * NSH]
    su = sgu[:, DFF * NSH:]
    sh = (sg * jax.nn.sigmoid(sg) * su).astype(jnp.bfloat16)
    out_ref[...] = jax.lax.dot_general(
        sh, sdn_ref[...].astype(jnp.bfloat16), (((1,), (1,)), ((), ())),
        preferred_element_type=jnp.float32)


def _shared_half(xh, sgub, sdnb):
    return pl.pallas_call(
        _shared_body,
        grid=(T // 2 // TME,),
        in_specs=[
            pl.BlockSpec((TME, D), lambda i: (i, 0)),
            pl.BlockSpec((2 * DFF * NSH, D), lambda i: (0, 0)),
            pl.BlockSpec((D, DFF * NSH), lambda i: (0, 0)),
        ],
        out_specs=pl.BlockSpec((TME, D), lambda i: (i, 0)),
        out_shape=jax.ShapeDtypeStruct((T // 2, D), jnp.float32),
        compiler_params=pltpu.CompilerParams(
            dimension_semantics=("arbitrary",),
        ),
    )(xh, sgub, sdnb)


def _combine_body(z0_ref, z1_ref, w0_ref, w1_ref, sh_ref, out_ref):
    out_ref[...] = (w0_ref[...] * z0_ref[...] + w1_ref[...] * z1_ref[...]
                    + sh_ref[...])


def _combine(z0, z1, w0, w1, shared_out):
    return pl.pallas_call(
        _combine_body,
        grid=(T // TME,),
        in_specs=[
            pl.BlockSpec((TME, D), lambda i: (i, 0)),
            pl.BlockSpec((TME, D), lambda i: (i, 0)),
            pl.BlockSpec((TME, 1), lambda i: (i, 0)),
            pl.BlockSpec((TME, 1), lambda i: (i, 0)),
            pl.BlockSpec((TME, D), lambda i: (i, 0)),
        ],
        out_specs=pl.BlockSpec((TME, D), lambda i: (i, 0)),
        out_shape=jax.ShapeDtypeStruct((T, D), jnp.float32),
        compiler_params=pltpu.CompilerParams(
            dimension_semantics=("arbitrary",),
        ),
    )(z0, z1, w0, w1, shared_out)


def kernel(hidden_states, residual, gate_weight, e_score_correction_bias,
           w13, w2, shared_gate_up, shared_down):
    del residual
    x = hidden_states
    gwp = jnp.zeros((EPAD, D), jnp.float32).at[:E].set(gate_weight)
    biasp = jnp.full((1, EPAD), -1e30, jnp.float32
                     ).at[0, :E].set(e_score_correction_bias)

    pos0, pos1, w0, w1, gid = _router(x, gwp, biasp)
    p0 = pos0.reshape(T)
    p1 = pos1.reshape(T)

    sgub = shared_gate_up
    sdnb = shared_down

    xs = _dispatch(x, p0, p1)
    sh_a = _shared_half(x[:T // 2], sgub, sdnb)
    ys = _gmm(gid.reshape(EPAD), xs, w13, w2)
    z0, z1 = _gatherz(ys, p0, p1)
    sh_b = _shared_half(x[T // 2:], sgub, sdnb)
    shared_out = jnp.concatenate([sh_a, sh_b], axis=0)

    return _combine(z0, z1, w0, w1, shared_out)


# speedup vs baseline: 1.2989x; 1.1569x over previous
import functools

import jax
import jax.numpy as jnp
from jax.experimental import pallas as pl
from jax.experimental.pallas import tpu as pltpu
from jax.experimental.pallas import tpu_sc as plsc

E = 8
TOPK = 2
D = 1024
DFF = 704
NSH = 2
T = 2048
RSF = 2.5
EPAD = 128

TM_C = 256
P = T * TOPK + E * TM_C
NT = P // TM_C
TME = 512

_SC_NC = 2
_SC_NS = 16
NW = _SC_NC * _SC_NS
TPW = T // NW


def _router_body(x_ref, gwp_ref, biasp_ref,
                 pos0_ref, pos1_ref, w0_ref, w1_ref, gid_ref):
    x = x_ref[...]
    logits = jax.lax.dot_general(
        x, gwp_ref[...], (((1,), (1,)), ((), ())),
        preferred_element_type=jnp.float32)
    s = jax.nn.sigmoid(logits)
    sel = s + biasp_ref[...]
    m1 = jnp.max(sel, axis=1, keepdims=True)
    t1 = sel >= m1
    sel2 = jnp.where(t1, -jnp.inf, sel)
    m2 = jnp.max(sel2, axis=1, keepdims=True)
    t2 = sel2 >= m2
    t1f = t1.astype(jnp.float32)
    t2f = t2.astype(jnp.float32)

    denom = jnp.sum(s * (t1f + t2f), axis=1, keepdims=True) + 1e-20
    w0_ref[...] = jnp.sum(s * t1f, axis=1, keepdims=True) * RSF / denom
    w1_ref[...] = jnp.sum(s * t2f, axis=1, keepdims=True) * RSF / denom

    cnt = t1f + t2f
    inc = cnt
    sft = 1
    while sft < T:
        inc = inc + jnp.concatenate(
            [jnp.zeros((sft, EPAD), jnp.float32), inc[:T - sft, :]], axis=0)
        sft *= 2
    exc = inc - cnt
    totals = inc[T - 1:T, :]

    tot_i = totals.astype(jnp.int32)
    lane = jax.lax.broadcasted_iota(jnp.int32, (1, EPAD), 1)
    jv = lane * TM_C
    offs = jnp.zeros((1, EPAD), jnp.float32)
    gid = jnp.zeros((1, EPAD), jnp.int32)
    run = jnp.zeros((), jnp.int32)
    for e in range(E):
        oh = lane == e
        te = jnp.sum(jnp.where(oh, tot_i, 0))
        pe = ((te + TM_C - 1) // TM_C) * TM_C
        offs = offs + jnp.where(oh, run, 0).astype(jnp.float32)
        run = run + pe
        gid = gid + (jv >= run).astype(jnp.int32)
    gid_ref[...] = jnp.minimum(gid, E - 1)

    dest = exc + offs
    pos0_ref[...] = jnp.sum(dest * t1f, axis=1, keepdims=True).astype(jnp.int32)
    pos1_ref[...] = jnp.sum(dest * t2f, axis=1, keepdims=True).astype(jnp.int32)


def _router(x, gwp, biasp):
    return pl.pallas_call(
        _router_body,
        out_shape=[
            jax.ShapeDtypeStruct((T, 1), jnp.int32),
            jax.ShapeDtypeStruct((T, 1), jnp.int32),
            jax.ShapeDtypeStruct((T, 1), jnp.float32),
            jax.ShapeDtypeStruct((T, 1), jnp.float32),
            jax.ShapeDtypeStruct((1, EPAD), jnp.int32),
        ],
    )(x, gwp, biasp)


def _dispatch_body(x_hbm, p0_hbm, p1_hbm, xs_hbm, idx_v, xv, sem):
    w = jax.lax.axis_index("s") * _SC_NC + jax.lax.axis_index("c")
    base = w * TPW
    pltpu.sync_copy(p0_hbm.at[pl.ds(base, TPW)], idx_v.at[0])
    pltpu.sync_copy(p1_hbm.at[pl.ds(base, TPW)], idx_v.at[1])
    pltpu.sync_copy(x_hbm.at[pl.ds(base, TPW)], xv)
    pltpu.async_copy(xv, xs_hbm.at[idx_v.at[0]], sem).wait()
    pltpu.async_copy(xv, xs_hbm.at[idx_v.at[1]], sem).wait()


@functools.lru_cache(maxsize=None)
def _make_dispatch():
    return pl.kernel(
        _dispatch_body,
        out_type=jax.ShapeDtypeStruct((P, D), jnp.float32),
        mesh=plsc.VectorSubcoreMesh(core_axis_name="c", subcore_axis_name="s"),
        scratch_types=[
            pltpu.VMEM((2, TPW), jnp.int32),
            pltpu.VMEM((TPW, D), jnp.float32),
            pltpu.SemaphoreType.DMA,
        ],
    )


def _dispatch(x, p0, p1):
    return _make_dispatch()(x, p0, p1)


def _gmm_body(gid_ref, xs_ref, w13_ref, w2_ref, ys_ref):
    xb = xs_ref[...].astype(jnp.bfloat16)
    gu = jax.lax.dot_general(
        xb, w13_ref[0].astype(jnp.bfloat16), (((1,), (1,)), ((), ())),
        preferred_element_type=jnp.float32)
    g = gu[:, :DFF]
    u = gu[:, DFF:]
    h = (g * jax.nn.sigmoid(g) * u).astype(jnp.bfloat16)
    ys_ref[...] = jax.lax.dot_general(
        h, w2_ref[0].astype(jnp.bfloat16), (((1,), (1,)), ((), ())),
        preferred_element_type=jnp.float32)


def _gmm(gid, xs, w13b, w2b):
    grid_spec = pltpu.PrefetchScalarGridSpec(
        num_scalar_prefetch=1,
        grid=(NT,),
        in_specs=[
            pl.BlockSpec((TM_C, D), lambda j, gid: (j, 0)),
            pl.BlockSpec((1, 2 * DFF, D), lambda j, gid: (gid[j], 0, 0)),
            pl.BlockSpec((1, D, DFF), lambda j, gid: (gid[j], 0, 0)),
        ],
        out_specs=pl.BlockSpec((TM_C, D), lambda j, gid: (j, 0)),
    )
    return pl.pallas_call(
        _gmm_body,
        grid_spec=grid_spec,
        out_shape=jax.ShapeDtypeStruct((P, D), jnp.float32),
        compiler_params=pltpu.CompilerParams(
            dimension_semantics=("arbitrary",),
        ),
    )(gid, xs, w13b, w2b)


def _gatherz_body(ys_hbm, p0_hbm, p1_hbm, z0_hbm, z1_hbm, idx_v, rv, sem):
    w = jax.lax.axis_index("s") * _SC_NC + jax.lax.axis_index("c")
    base = w * TPW
    pltpu.sync_copy(p0_hbm.at[pl.ds(base, TPW)], idx_v.at[0])
    pltpu.sync_copy(p1_hbm.at[pl.ds(base, TPW)], idx_v.at[1])
    pltpu.async_copy(ys_hbm.at[idx_v.at[0]], rv, sem).wait()
    pltpu.sync_copy(rv, z0_hbm.at[pl.ds(base, TPW)])
    pltpu.async_copy(ys_hbm.at[idx_v.at[1]], rv, sem).wait()
    pltpu.sync_copy(rv, z1_hbm.at[pl.ds(base, TPW)])


@functools.lru_cache(maxsize=None)
def _make_gatherz():
    return pl.kernel(
        _gatherz_body,
        out_type=(
            jax.ShapeDtypeStruct((T, D), jnp.float32),
            jax.ShapeDtypeStruct((T, D), jnp.float32),
        ),
        mesh=plsc.VectorSubcoreMesh(core_axis_name="c", subcore_axis_name="s"),
        scratch_types=[
            pltpu.VMEM((2, TPW), jnp.int32),
            pltpu.VMEM((TPW, D), jnp.float32),
            pltpu.SemaphoreType.DMA,
        ],
    )


def _gatherz(ys, p0, p1):
    return _make_gatherz()(ys, p0, p1)


def _shared_body(x_ref, sgu_ref, sdn_ref, out_ref):
    xb = x_ref[...].astype(jnp.bfloat16)
    sgu = jax.lax.dot_general(
        xb, sgu_ref[...].astype(jnp.bfloat16), (((1,), (1,)), ((), ())),
        preferred_element_type=jnp.float32)
    sg = sgu[:, :DFF * NSH]
    su = sgu[:, DFF * NSH:]
    sh = (sg * jax.nn.sigmoid(sg) * su).astype(jnp.bfloat16)
    out_ref[...] = jax.lax.dot_general(
        sh, sdn_ref[...].astype(jnp.bfloat16), (((1,), (1,)), ((), ())),
        preferred_element_type=jnp.float32)


def _combine_body(z0_ref, z1_ref, w0_ref, w1_ref, x_ref, sgu_ref, sdn_ref,
                  out_ref):
    xb = x_ref[...].astype(jnp.bfloat16)
    sgu = jax.lax.dot_general(
        xb, sgu_ref[...].astype(jnp.bfloat16), (((1,), (1,)), ((), ())),
        preferred_element_type=jnp.float32)
    sg = sgu[:, :DFF * NSH]
    su = sgu[:, DFF * NSH:]
    sh = (sg * jax.nn.sigmoid(sg) * su).astype(jnp.bfloat16)
    shared = jax.lax.dot_general(
        sh, sdn_ref[...].astype(jnp.bfloat16), (((1,), (1,)), ((), ())),
        preferred_element_type=jnp.float32)
    out_ref[...] = (w0_ref[...] * z0_ref[...] + w1_ref[...] * z1_ref[...]
                    + shared)


def _combine(z0, z1, w0, w1, x, sgu_w, sdn_w):
    return pl.pallas_call(
        _combine_body,
        grid=(T // TME,),
        in_specs=[
            pl.BlockSpec((TME, D), lambda i: (i, 0)),
            pl.BlockSpec((TME, D), lambda i: (i, 0)),
            pl.BlockSpec((TME, 1), lambda i: (i, 0)),
            pl.BlockSpec((TME, 1), lambda i: (i, 0)),
            pl.BlockSpec((TME, D), lambda i: (i, 0)),
            pl.BlockSpec((2 * DFF * NSH, D), lambda i: (0, 0)),
            pl.BlockSpec((D, DFF * NSH), lambda i: (0, 0)),
        ],
        out_specs=pl.BlockSpec((TME, D), lambda i: (i, 0)),
        out_shape=jax.ShapeDtypeStruct((T, D), jnp.float32),
        compiler_params=pltpu.CompilerParams(
            dimension_semantics=("arbitrary",),
        ),
    )(z0, z1, w0, w1, x, sgu_w, sdn_w)


def kernel(hidden_states, residual, gate_weight, e_score_correction_bias,
           w13, w2, shared_gate_up, shared_down):
    del residual
    x = hidden_states
    gwp = jnp.zeros((EPAD, D), jnp.float32).at[:E].set(gate_weight)
    biasp = jnp.full((1, EPAD), -1e30, jnp.float32
                     ).at[0, :E].set(e_score_correction_bias)

    pos0, pos1, w0, w1, gid = _router(x, gwp, biasp)
    p0 = pos0.reshape(T)
    p1 = pos1.reshape(T)

    xs = _dispatch(x, p0, p1)
    ys = _gmm(gid.reshape(EPAD), xs, w13, w2)
    z0, z1 = _gatherz(ys, p0, p1)

    return _combine(z0, z1, w0, w1, x, shared_gate_up, shared_down)


# speedup vs baseline: 1.3221x; 1.0178x over previous
import functools

import jax
import jax.numpy as jnp
from jax.experimental import pallas as pl
from jax.experimental.pallas import tpu as pltpu
from jax.experimental.pallas import tpu_sc as plsc

E = 8
TOPK = 2
D = 1024
DFF = 704
NSH = 2
T = 2048
RSF = 2.5
EPAD = 128

TM_C = 256
P = T * TOPK + E * TM_C
NT = P // TM_C
TME = 512

_SC_NC = 2
_SC_NS = 16
NW = _SC_NC * _SC_NS
TPW = T // NW


def _router_body(x_ref, gwp_ref, biasp_ref,
                 pos0_ref, pos1_ref, w0_ref, w1_ref, gid_ref, valid_ref):
    x = x_ref[...]
    logits = jax.lax.dot_general(
        x, gwp_ref[...], (((1,), (1,)), ((), ())),
        preferred_element_type=jnp.float32)
    s = jax.nn.sigmoid(logits)
    sel = s + biasp_ref[...]
    m1 = jnp.max(sel, axis=1, keepdims=True)
    t1 = sel >= m1
    sel2 = jnp.where(t1, -jnp.inf, sel)
    m2 = jnp.max(sel2, axis=1, keepdims=True)
    t2 = sel2 >= m2
    t1f = t1.astype(jnp.float32)
    t2f = t2.astype(jnp.float32)

    denom = jnp.sum(s * (t1f + t2f), axis=1, keepdims=True) + 1e-20
    w0_ref[...] = jnp.sum(s * t1f, axis=1, keepdims=True) * RSF / denom
    w1_ref[...] = jnp.sum(s * t2f, axis=1, keepdims=True) * RSF / denom

    cnt = t1f + t2f
    inc = cnt
    sft = 1
    while sft < T:
        inc = inc + jnp.concatenate(
            [jnp.zeros((sft, EPAD), jnp.float32), inc[:T - sft, :]], axis=0)
        sft *= 2
    exc = inc - cnt
    totals = inc[T - 1:T, :]

    tot_i = totals.astype(jnp.int32)
    lane = jax.lax.broadcasted_iota(jnp.int32, (1, EPAD), 1)
    jv = lane * TM_C
    offs = jnp.zeros((1, EPAD), jnp.float32)
    gid = jnp.zeros((1, EPAD), jnp.int32)
    run = jnp.zeros((), jnp.int32)
    for e in range(E):
        oh = lane == e
        te = jnp.sum(jnp.where(oh, tot_i, 0))
        pe = ((te + TM_C - 1) // TM_C) * TM_C
        offs = offs + jnp.where(oh, run, 0).astype(jnp.float32)
        run = run + pe
        gid = gid + (jv >= run).astype(jnp.int32)
    gid_ref[...] = jnp.minimum(gid, E - 1)
    valid_ref[...] = (jv < run).astype(jnp.int32)

    dest = exc + offs
    pos0_ref[...] = jnp.sum(dest * t1f, axis=1, keepdims=True).astype(jnp.int32)
    pos1_ref[...] = jnp.sum(dest * t2f, axis=1, keepdims=True).astype(jnp.int32)


def _router(x, gwp, biasp):
    return pl.pallas_call(
        _router_body,
        out_shape=[
            jax.ShapeDtypeStruct((T, 1), jnp.int32),
            jax.ShapeDtypeStruct((T, 1), jnp.int32),
            jax.ShapeDtypeStruct((T, 1), jnp.float32),
            jax.ShapeDtypeStruct((T, 1), jnp.float32),
            jax.ShapeDtypeStruct((1, EPAD), jnp.int32),
            jax.ShapeDtypeStruct((1, EPAD), jnp.int32),
        ],
    )(x, gwp, biasp)


def _dispatch_body(x_hbm, p0_hbm, p1_hbm, xs_hbm, idx_v, xv, sem):
    w = jax.lax.axis_index("s") * _SC_NC + jax.lax.axis_index("c")
    base = w * TPW
    pltpu.sync_copy(p0_hbm.at[pl.ds(base, TPW)], idx_v.at[0])
    pltpu.sync_copy(p1_hbm.at[pl.ds(base, TPW)], idx_v.at[1])
    pltpu.sync_copy(x_hbm.at[pl.ds(base, TPW)], xv)
    pltpu.async_copy(xv, xs_hbm.at[idx_v.at[0]], sem).wait()
    pltpu.async_copy(xv, xs_hbm.at[idx_v.at[1]], sem).wait()


@functools.lru_cache(maxsize=None)
def _make_dispatch():
    return pl.kernel(
        _dispatch_body,
        out_type=jax.ShapeDtypeStruct((P, D), jnp.float32),
        mesh=plsc.VectorSubcoreMesh(core_axis_name="c", subcore_axis_name="s"),
        scratch_types=[
            pltpu.VMEM((2, TPW), jnp.int32),
            pltpu.VMEM((TPW, D), jnp.float32),
            pltpu.SemaphoreType.DMA,
        ],
    )


def _dispatch(x, p0, p1):
    return _make_dispatch()(x, p0, p1)


def _gmm_body(gid_ref, valid_ref, xs_ref, w13_ref, w2_ref, ys_ref):
    j = pl.program_id(0)

    @pl.when(valid_ref[j] > 0)
    def _tile():
        xb = xs_ref[...].astype(jnp.bfloat16)
        gu = jax.lax.dot_general(
            xb, w13_ref[0].astype(jnp.bfloat16), (((1,), (1,)), ((), ())),
            preferred_element_type=jnp.float32)
        g = gu[:, :DFF]
        u = gu[:, DFF:]
        h = (g * jax.nn.sigmoid(g) * u).astype(jnp.bfloat16)
        ys_ref[...] = jax.lax.dot_general(
            h, w2_ref[0].astype(jnp.bfloat16), (((1,), (1,)), ((), ())),
            preferred_element_type=jnp.float32)


def _gmm(gid, valid, xs, w13b, w2b):
    grid_spec = pltpu.PrefetchScalarGridSpec(
        num_scalar_prefetch=2,
        grid=(NT,),
        in_specs=[
            pl.BlockSpec((TM_C, D), lambda j, gid, valid: (j, 0)),
            pl.BlockSpec((1, 2 * DFF, D), lambda j, gid, valid: (gid[j], 0, 0)),
            pl.BlockSpec((1, D, DFF), lambda j, gid, valid: (gid[j], 0, 0)),
        ],
        out_specs=pl.BlockSpec((TM_C, D), lambda j, gid, valid: (j, 0)),
    )
    return pl.pallas_call(
        _gmm_body,
        grid_spec=grid_spec,
        out_shape=jax.ShapeDtypeStruct((P, D), jnp.float32),
        compiler_params=pltpu.CompilerParams(
            dimension_semantics=("arbitrary",),
        ),
    )(gid, valid, xs, w13b, w2b)


def _gatherz_body(ys_hbm, p0_hbm, p1_hbm, z0_hbm, z1_hbm, idx_v, rv, sem):
    w = jax.lax.axis_index("s") * _SC_NC + jax.lax.axis_index("c")
    base = w * TPW
    pltpu.sync_copy(p0_hbm.at[pl.ds(base, TPW)], idx_v.at[0])
    pltpu.sync_copy(p1_hbm.at[pl.ds(base, TPW)], idx_v.at[1])
    pltpu.async_copy(ys_hbm.at[idx_v.at[0]], rv, sem).wait()
    pltpu.sync_copy(rv, z0_hbm.at[pl.ds(base, TPW)])
    pltpu.async_copy(ys_hbm.at[idx_v.at[1]], rv, sem).wait()
    pltpu.sync_copy(rv, z1_hbm.at[pl.ds(base, TPW)])


@functools.lru_cache(maxsize=None)
def _make_gatherz():
    return pl.kernel(
        _gatherz_body,
        out_type=(
            jax.ShapeDtypeStruct((T, D), jnp.float32),
            jax.ShapeDtypeStruct((T, D), jnp.float32),
        ),
        mesh=plsc.VectorSubcoreMesh(core_axis_name="c", subcore_axis_name="s"),
        scratch_types=[
            pltpu.VMEM((2, TPW), jnp.int32),
            pltpu.VMEM((TPW, D), jnp.float32),
            pltpu.SemaphoreType.DMA,
        ],
    )


def _gatherz(ys, p0, p1):
    return _make_gatherz()(ys, p0, p1)


def _shared_body(x_ref, sgu_ref, sdn_ref, out_ref):
    xb = x_ref[...].astype(jnp.bfloat16)
    sgu = jax.lax.dot_general(
        xb, sgu_ref[...].astype(jnp.bfloat16), (((1,), (1,)), ((), ())),
        preferred_element_type=jnp.float32)
    sg = sgu[:, :DFF * NSH]
    su = sgu[:, DFF * NSH:]
    sh = (sg * jax.nn.sigmoid(sg) * su).astype(jnp.bfloat16)
    out_ref[...] = jax.lax.dot_general(
        sh, sdn_ref[...].astype(jnp.bfloat16), (((1,), (1,)), ((), ())),
        preferred_element_type=jnp.float32)


def _combine_body(z0_ref, z1_ref, w0_ref, w1_ref, x_ref, sgu_ref, sdn_ref,
                  out_ref):
    xb = x_ref[...].astype(jnp.bfloat16)
    sgu = jax.lax.dot_general(
        xb, sgu_ref[...].astype(jnp.bfloat16), (((1,), (1,)), ((), ())),
        preferred_element_type=jnp.float32)
    sg = sgu[:, :DFF * NSH]
    su = sgu[:, DFF * NSH:]
    sh = (sg * jax.nn.sigmoid(sg) * su).astype(jnp.bfloat16)
    shared = jax.lax.dot_general(
        sh, sdn_ref[...].astype(jnp.bfloat16), (((1,), (1,)), ((), ())),
        preferred_element_type=jnp.float32)
    out_ref[...] = (w0_ref[...] * z0_ref[...] + w1_ref[...] * z1_ref[...]
                    + shared)


def _combine(z0, z1, w0, w1, x, sgu_w, sdn_w):
    return pl.pallas_call(
        _combine_body,
        grid=(T // TME,),
        in_specs=[
            pl.BlockSpec((TME, D), lambda i: (i, 0)),
            pl.BlockSpec((TME, D), lambda i: (i, 0)),
            pl.BlockSpec((TME, 1), lambda i: (i, 0)),
            pl.BlockSpec((TME, 1), lambda i: (i, 0)),
            pl.BlockSpec((TME, D), lambda i: (i, 0)),
            pl.BlockSpec((2 * DFF * NSH, D), lambda i: (0, 0)),
            pl.BlockSpec((D, DFF * NSH), lambda i: (0, 0)),
        ],
        out_specs=pl.BlockSpec((TME, D), lambda i: (i, 0)),
        out_shape=jax.ShapeDtypeStruct((T, D), jnp.float32),
        compiler_params=pltpu.CompilerParams(
            dimension_semantics=("arbitrary",),
        ),
    )(z0, z1, w0, w1, x, sgu_w, sdn_w)


def kernel(hidden_states, residual, gate_weight, e_score_correction_bias,
           w13, w2, shared_gate_up, shared_down):
    del residual
    x = hidden_states
    gwp = jnp.zeros((EPAD, D), jnp.float32).at[:E].set(gate_weight)
    biasp = jnp.full((1, EPAD), -1e30, jnp.float32
                     ).at[0, :E].set(e_score_correction_bias)

    pos0, pos1, w0, w1, gid, valid = _router(x, gwp, biasp)
    p0 = pos0.reshape(T)
    p1 = pos1.reshape(T)

    xs = _dispatch(x, p0, p1)
    ys = _gmm(gid.reshape(EPAD), valid.reshape(EPAD), xs, w13, w2)
    z0, z1 = _gatherz(ys, p0, p1)

    return _combine(z0, z1, w0, w1, x, shared_gate_up, shared_down)


# speedup vs baseline: 1.3244x; 1.0017x over previous
import functools

import jax
import jax.numpy as jnp
from jax.experimental import pallas as pl
from jax.experimental.pallas import tpu as pltpu
from jax.experimental.pallas import tpu_sc as plsc

E = 8
TOPK = 2
D = 1024
DFF = 704
NSH = 2
T = 2048
RSF = 2.5
EPAD = 128

TM_C = 256
P = T * TOPK + E * TM_C
NT = P // TM_C
TME = 512

_SC_NC = 2
_SC_NS = 16
NW = _SC_NC * _SC_NS
TPW = T // NW


def _router_body(x_ref, gwp_ref, biasp_ref,
                 pos0_ref, pos1_ref, w0_ref, w1_ref, gid_ref, valid_ref):
    x = x_ref[...]
    logits = jax.lax.dot_general(
        x, gwp_ref[...], (((1,), (1,)), ((), ())),
        preferred_element_type=jnp.float32)
    s = jax.nn.sigmoid(logits)
    sel = s + biasp_ref[...]
    m1 = jnp.max(sel, axis=1, keepdims=True)
    t1 = sel >= m1
    sel2 = jnp.where(t1, -jnp.inf, sel)
    m2 = jnp.max(sel2, axis=1, keepdims=True)
    t2 = sel2 >= m2
    t1f = t1.astype(jnp.float32)
    t2f = t2.astype(jnp.float32)

    denom = jnp.sum(s * (t1f + t2f), axis=1, keepdims=True) + 1e-20
    w0_ref[...] = jnp.sum(s * t1f, axis=1, keepdims=True) * RSF / denom
    w1_ref[...] = jnp.sum(s * t2f, axis=1, keepdims=True) * RSF / denom

    cnt = t1f + t2f
    inc = cnt
    sft = 1
    while sft < T:
        inc = inc + jnp.concatenate(
            [jnp.zeros((sft, EPAD), jnp.float32), inc[:T - sft, :]], axis=0)
        sft *= 2
    exc = inc - cnt
    totals = inc[T - 1:T, :]

    tot_i = totals.astype(jnp.int32)
    lane = jax.lax.broadcasted_iota(jnp.int32, (1, EPAD), 1)
    jv = lane * TM_C
    offs = jnp.zeros((1, EPAD), jnp.float32)
    gid = jnp.zeros((1, EPAD), jnp.int32)
    run = jnp.zeros((), jnp.int32)
    for e in range(E):
        oh = lane == e
        te = jnp.sum(jnp.where(oh, tot_i, 0))
        pe = ((te + TM_C - 1) // TM_C) * TM_C
        offs = offs + jnp.where(oh, run, 0).astype(jnp.float32)
        run = run + pe
        gid = gid + (jv >= run).astype(jnp.int32)
    gid_ref[...] = jnp.minimum(gid, E - 1)
    valid_ref[...] = (jv < run).astype(jnp.int32)

    dest = exc + offs
    pos0_ref[...] = jnp.sum(dest * t1f, axis=1, keepdims=True).astype(jnp.int32)
    pos1_ref[...] = jnp.sum(dest * t2f, axis=1, keepdims=True).astype(jnp.int32)


def _router(x, gwp, biasp):
    return pl.pallas_call(
        _router_body,
        out_shape=[
            jax.ShapeDtypeStruct((T, 1), jnp.int32),
            jax.ShapeDtypeStruct((T, 1), jnp.int32),
            jax.ShapeDtypeStruct((T, 1), jnp.float32),
            jax.ShapeDtypeStruct((T, 1), jnp.float32),
            jax.ShapeDtypeStruct((1, EPAD), jnp.int32),
            jax.ShapeDtypeStruct((1, EPAD), jnp.int32),
        ],
    )(x, gwp, biasp)


def _dispatch_body(x_hbm, p0_hbm, p1_hbm, xs_hbm, idx_v, xv, sem):
    w = jax.lax.axis_index("s") * _SC_NC + jax.lax.axis_index("c")
    base = w * TPW
    pltpu.sync_copy(p0_hbm.at[pl.ds(base, TPW)], idx_v.at[0])
    pltpu.sync_copy(p1_hbm.at[pl.ds(base, TPW)], idx_v.at[1])
    pltpu.sync_copy(x_hbm.at[pl.ds(base, TPW)], xv)
    pltpu.async_copy(xv, xs_hbm.at[idx_v.at[0]], sem).wait()
    pltpu.async_copy(xv, xs_hbm.at[idx_v.at[1]], sem).wait()


@functools.lru_cache(maxsize=None)
def _make_dispatch():
    return pl.kernel(
        _dispatch_body,
        out_type=jax.ShapeDtypeStruct((P, D), jnp.float32),
        mesh=plsc.VectorSubcoreMesh(core_axis_name="c", subcore_axis_name="s"),
        scratch_types=[
            pltpu.VMEM((2, TPW), jnp.int32),
            pltpu.VMEM((TPW, D), jnp.float32),
            pltpu.SemaphoreType.DMA,
        ],
    )


def _dispatch(x, p0, p1):
    return _make_dispatch()(x, p0, p1)


def _gmm_body(gid_ref, valid_ref, xs_ref, w13_ref, w2_ref, ys_ref):
    j = pl.program_id(0)

    @pl.when(valid_ref[j] > 0)
    def _tile():
        xb = xs_ref[...].astype(jnp.bfloat16)
        gu = jax.lax.dot_general(
            xb, w13_ref[0].astype(jnp.bfloat16), (((1,), (1,)), ((), ())),
            preferred_element_type=jnp.float32)
        g = gu[:, :DFF]
        u = gu[:, DFF:]
        h = (g * jax.nn.sigmoid(g) * u).astype(jnp.bfloat16)
        ys_ref[...] = jax.lax.dot_general(
            h, w2_ref[0].astype(jnp.bfloat16), (((1,), (1,)), ((), ())),
            preferred_element_type=jnp.float32)


def _gmm(gid, valid, xs, w13b, w2b):
    grid_spec = pltpu.PrefetchScalarGridSpec(
        num_scalar_prefetch=2,
        grid=(NT,),
        in_specs=[
            pl.BlockSpec((TM_C, D), lambda j, gid, valid: (j, 0)),
            pl.BlockSpec((1, 2 * DFF, D), lambda j, gid, valid: (gid[j], 0, 0)),
            pl.BlockSpec((1, D, DFF), lambda j, gid, valid: (gid[j], 0, 0)),
        ],
        out_specs=pl.BlockSpec((TM_C, D), lambda j, gid, valid: (j, 0)),
    )
    return pl.pallas_call(
        _gmm_body,
        grid_spec=grid_spec,
        out_shape=jax.ShapeDtypeStruct((P, D), jnp.float32),
        compiler_params=pltpu.CompilerParams(
            dimension_semantics=("arbitrary",),
        ),
    )(gid, valid, xs, w13b, w2b)


def _gatherz_body(ys_hbm, p0_hbm, p1_hbm, z0_hbm, z1_hbm, idx_v, rv, sem):
    w = jax.lax.axis_index("s") * _SC_NC + jax.lax.axis_index("c")
    base = w * TPW
    pltpu.sync_copy(p0_hbm.at[pl.ds(base, TPW)], idx_v.at[0])
    pltpu.sync_copy(p1_hbm.at[pl.ds(base, TPW)], idx_v.at[1])
    pltpu.async_copy(ys_hbm.at[idx_v.at[0]], rv, sem).wait()
    pltpu.sync_copy(rv, z0_hbm.at[pl.ds(base, TPW)])
    pltpu.async_copy(ys_hbm.at[idx_v.at[1]], rv, sem).wait()
    pltpu.sync_copy(rv, z1_hbm.at[pl.ds(base, TPW)])


@functools.lru_cache(maxsize=None)
def _make_gatherz():
    return pl.kernel(
        _gatherz_body,
        out_type=(
            jax.ShapeDtypeStruct((T, D), jnp.float32),
            jax.ShapeDtypeStruct((T, D), jnp.float32),
        ),
        mesh=plsc.VectorSubcoreMesh(core_axis_name="c", subcore_axis_name="s"),
        scratch_types=[
            pltpu.VMEM((2, TPW), jnp.int32),
            pltpu.VMEM((TPW, D), jnp.float32),
            pltpu.SemaphoreType.DMA,
        ],
    )


def _gatherz(ys, p0, p1):
    return _make_gatherz()(ys, p0, p1)


def _combine_body(z0_ref, z1_ref, w0_ref, w1_ref, x_ref, sgu_ref, sdn_ref,
                  out_ref):
    xb = x_ref[...].astype(jnp.bfloat16)
    sgu = jax.lax.dot_general(
        xb, sgu_ref[...].astype(jnp.bfloat16), (((1,), (1,)), ((), ())),
        preferred_element_type=jnp.float32)
    sg = sgu[:, :DFF * NSH]
    su = sgu[:, DFF * NSH:]
    sh = (sg * jax.nn.sigmoid(sg) * su).astype(jnp.bfloat16)
    shared = jax.lax.dot_general(
        sh, sdn_ref[...].astype(jnp.bfloat16), (((1,), (1,)), ((), ())),
        preferred_element_type=jnp.float32)
    out_ref[...] = (w0_ref[...] * z0_ref[...] + w1_ref[...] * z1_ref[...]
                    + shared)


def _combine(z0, z1, w0, w1, x, sgu_w, sdn_w):
    return pl.pallas_call(
        _combine_body,
        grid=(T // TME,),
        in_specs=[
            pl.BlockSpec((TME, D), lambda i: (i, 0)),
            pl.BlockSpec((TME, D), lambda i: (i, 0)),
            pl.BlockSpec((TME, 1), lambda i: (i, 0)),
            pl.BlockSpec((TME, 1), lambda i: (i, 0)),
            pl.BlockSpec((TME, D), lambda i: (i, 0)),
            pl.BlockSpec((2 * DFF * NSH, D), lambda i: (0, 0)),
            pl.BlockSpec((D, DFF * NSH), lambda i: (0, 0)),
        ],
        out_specs=pl.BlockSpec((TME, D), lambda i: (i, 0)),
        out_shape=jax.ShapeDtypeStruct((T, D), jnp.float32),
        compiler_params=pltpu.CompilerParams(
            dimension_semantics=("arbitrary",),
        ),
    )(z0, z1, w0, w1, x, sgu_w, sdn_w)


def kernel(hidden_states, residual, gate_weight, e_score_correction_bias,
           w13, w2, shared_gate_up, shared_down):
    del residual
    x = hidden_states
    gwp = jnp.zeros((EPAD, D), jnp.float32).at[:E].set(gate_weight)
    biasp = jnp.full((1, EPAD), -1e30, jnp.float32
                     ).at[0, :E].set(e_score_correction_bias)

    pos0, pos1, w0, w1, gid, valid = _router(x, gwp, biasp)
    p0 = pos0.reshape(T)
    p1 = pos1.reshape(T)

    xs = _dispatch(x, p0, p1)
    ys = _gmm(gid.reshape(EPAD), valid.reshape(EPAD), xs, w13, w2)
    z0, z1 = _gatherz(ys, p0, p1)

    return _combine(z0, z1, w0, w1, x, shared_gate_up, shared_down)
